# baseline (speedup 1.0000x reference)
import jax
import jax.numpy as jnp
from jax import lax
from jax.experimental import pallas as pl
from jax.experimental.pallas import tpu as pltpu
from jax.experimental.pallas import tpu_sc as plsc

VOCAB = 100000
HIDDEN = 128

NC = 2
NS = 16
NW = NC * NS

BATCH = 4096
SEQ = 50
N_TOKENS = BATCH * SEQ
IDX_COLS = 128
N_IDX_ROWS = N_TOKENS // IDX_COLS
ROWS_PER_W = N_IDX_ROWS // NW

RING = 5


def _body(tab_hbm, ids_hbm, out_hbm, idx_v, rows_v, gsem, wsem):
    c = lax.axis_index("c")
    s = lax.axis_index("s")
    wid = s * NC + c
    row0 = wid * ROWS_PER_W

    pltpu.sync_copy(ids_hbm.at[wid], idx_v)

    def out_ref(j):
        return out_hbm.at[pl.ds((row0 + j) * IDX_COLS, IDX_COLS)]

    def g(j, b):
        return pltpu.make_async_copy(tab_hbm.at[idx_v.at[j]], rows_v.at[b], gsem.at[b])

    def w(j, b):
        return pltpu.make_async_copy(rows_v.at[b], out_ref(j), wsem.at[b])

    for b in range(RING):
        g(b, b).start()

    def step(j, carry):
        b = lax.rem(j, RING)
        g(j, b).wait()
        w(j, b).start()
        w(j, b).wait()
        g(j + RING, b).start()
        return carry

    lax.fori_loop(0, ROWS_PER_W - RING, step, 0)

    def tail(j, carry):
        b = lax.rem(j, RING)
        g(j, b).wait()
        w(j, b).start()
        w(j, b).wait()
        return carry

    lax.fori_loop(ROWS_PER_W - RING, ROWS_PER_W, tail, 0)


@jax.jit
def _embed(table, ids_t3):
    mesh = plsc.VectorSubcoreMesh(core_axis_name="c", subcore_axis_name="s")
    f = pl.kernel(
        _body,
        out_type=jax.ShapeDtypeStruct((N_TOKENS, HIDDEN), jnp.float32),
        name="sc_embedding_gather",
        mesh=mesh,
        scratch_types=[
            pltpu.VMEM((ROWS_PER_W, IDX_COLS), jnp.int32),
            pltpu.VMEM((RING, IDX_COLS, HIDDEN), jnp.float32),
            pltpu.SemaphoreType.DMA((RING,)),
            pltpu.SemaphoreType.DMA((RING,)),
        ],
    )
    out = f(table, ids_t3)
    return out.reshape(SEQ, BATCH, HIDDEN).transpose(1, 0, 2)


def kernel(input_ids, table):
    ids_t3 = input_ids.T.reshape(NW, ROWS_PER_W, IDX_COLS)
    return _embed(table, ids_t3)


# speedup vs baseline: 1.0074x; 1.0074x over previous
import jax
import jax.numpy as jnp
from jax import lax
from jax.experimental import pallas as pl
from jax.experimental.pallas import tpu as pltpu
from jax.experimental.pallas import tpu_sc as plsc

VOCAB = 100000
HIDDEN = 128

NC = 2
NS = 16
NW = NC * NS

BATCH = 4096
SEQ = 50
N_TOKENS = BATCH * SEQ
IDX_COLS = 128
N_IDX_ROWS = N_TOKENS // IDX_COLS
ROWS_PER_W = N_IDX_ROWS // NW

RING = 5
DEPTH = 2
N_GROUPS = ROWS_PER_W // RING


def _body(tab_hbm, ids_hbm, out_hbm, idx_v, rows_v, gsem, wsem):
    c = lax.axis_index("c")
    s = lax.axis_index("s")
    wid = s * NC + c
    row0 = wid * ROWS_PER_W

    pltpu.sync_copy(ids_hbm.at[wid], idx_v)

    def out_ref(j):
        return out_hbm.at[pl.ds((row0 + j) * IDX_COLS, IDX_COLS)]

    def start_gather(j, b):
        pltpu.make_async_copy(tab_hbm.at[idx_v.at[j]], rows_v.at[b], gsem.at[b]).start()

    def wait_gather(j, b):
        pltpu.make_async_copy(tab_hbm.at[idx_v.at[j]], rows_v.at[b], gsem.at[b]).wait()

    def start_wb(j, b):
        pltpu.make_async_copy(rows_v.at[b], out_ref(j), wsem.at[b]).start()

    def wait_wb(j, b):
        pltpu.make_async_copy(rows_v.at[b], out_ref(j), wsem.at[b]).wait()

    for j in range(DEPTH):
        start_gather(j, j)

    for b in range(RING):
        j = b
        wait_gather(j, b)
        start_wb(j, b)
        jn = j + DEPTH
        bn = jn % RING
        if jn >= RING:
            wait_wb(jn - RING, bn)
        start_gather(jn, bn)

    def group(g, carry):
        for b in range(RING):
            j = g * RING + b
            bn = (b + DEPTH) % RING
            wait_gather(j, b)
            start_wb(j, b)
            wait_wb(j + DEPTH - RING, bn)
            start_gather(j + DEPTH, bn)
        return carry

    lax.fori_loop(1, N_GROUPS - 1, group, 0)

    for b in range(RING):
        j = (N_GROUPS - 1) * RING + b
        wait_gather(j, b)
        start_wb(j, b)
        jn = j + DEPTH
        if jn < ROWS_PER_W:
            bn = (b + DEPTH) % RING
            wait_wb(jn - RING, bn)
            start_gather(jn, bn)
    for b in range(RING):
        wait_wb((N_GROUPS - 1) * RING + b, b)


@jax.jit
def _embed(table, ids_t3):
    mesh = plsc.VectorSubcoreMesh(core_axis_name="c", subcore_axis_name="s")
    f = pl.kernel(
        _body,
        out_type=jax.ShapeDtypeStruct((N_TOKENS, HIDDEN), jnp.float32),
        name="sc_embedding_gather",
        mesh=mesh,
        scratch_types=[
            pltpu.VMEM((ROWS_PER_W, IDX_COLS), jnp.int32),
            pltpu.VMEM((RING, IDX_COLS, HIDDEN), jnp.float32),
            pltpu.SemaphoreType.DMA((RING,)),
            pltpu.SemaphoreType.DMA((RING,)),
        ],
    )
    out = f(table, ids_t3)
    return out.reshape(SEQ, BATCH, HIDDEN).transpose(1, 0, 2)


def kernel(input_ids, table):
    ids_t3 = input_ids.T.reshape(NW, ROWS_PER_W, IDX_COLS)
    return _embed(table, ids_t3)
